# baseline (speedup 1.0000x reference)
import numpy as np
import jax
import jax.numpy as jnp
from jax.experimental import pallas as pl

B, C, L = 8, 384, 576
M = 1536
H = 8
TOPK = 128
D = C // H
BH = B * H

_HI = jax.lax.Precision.HIGHEST
_DEF = jax.lax.Precision.DEFAULT


def _dot_nt(a, b, precision=_DEF):
    return jax.lax.dot_general(a, b, (((1,), (1,)), ((), ())),
                               precision=precision,
                               preferred_element_type=jnp.float32)


def _dot_nn(a, b, precision=_HI):
    return jax.lax.dot_general(a, b, (((1,), (0,)), ((), ())),
                               precision=precision,
                               preferred_element_type=jnp.float32)


def _dot_nn_fast(a, b, precision=_DEF):
    return jax.lax.dot_general(a, b, (((1,), (0,)), ((), ())),
                               precision=precision,
                               preferred_element_type=jnp.float32)


def _bdot_nt(a, b):
    return jax.lax.dot_general(a.astype(jnp.bfloat16), b.astype(jnp.bfloat16),
                               (((1,), (1,)), ((), ())),
                               preferred_element_type=jnp.float32)


def _bdot_nn(a, b):
    return jax.lax.dot_general(a.astype(jnp.bfloat16), b.astype(jnp.bfloat16),
                               (((1,), (0,)), ((), ())),
                               preferred_element_type=jnp.float32)


def _gn_scalar(x, g, b):
    mu = jnp.mean(x)
    var = jnp.mean((x - mu) * (x - mu))
    return (x - mu) * jax.lax.rsqrt(var + 1e-5) * g + b


def _topk_kernel(s_ref, g_ref):
    act = s_ref[...]
    iota_l = jax.lax.broadcasted_iota(jnp.int32, (BH, L), 1)
    cols = []
    for _t in range(TOPK):
        m = jnp.max(act, axis=1, keepdims=True)
        first = jnp.min(jnp.where(act == m, iota_l, L + 1),
                        axis=1, keepdims=True)
        cols.append(first)
        act = jnp.where(iota_l == first, -jnp.inf, act)
    g_ref[...] = jnp.concatenate(cols, axis=1)


BPP = 1


def _block_kernel(g_ref, q_ref, k_ref, v_ref, x_ref,
                  wq_ref, wk_ref, wv_ref, bq_ref, bk_ref, bv_ref,
                  bias_k_ref, bias_v_ref, out_w_ref, out_b_ref,
                  an_g_ref, an_b_ref, gamma_ref,
                  m1_w_ref, m1_b_ref, m1_ng_ref, m1_nb_ref,
                  m2_w_ref, m2_b_ref, m2_ng_ref, m2_nb_ref,
                  out_ref):
    iota_tl = jax.lax.broadcasted_iota(jnp.int32, (TOPK, L), 1)
    scale = jnp.float32(1.0 / np.sqrt(D))

    for bi in range(BPP):
        gidx = g_ref[bi]
        qc = q_ref[bi]
        kcol = k_ref[bi]
        Vn = v_ref[bi]
        xn = x_ref[bi]

        kc_parts, vc_parts = [], []
        for h in range(H):
            sl = slice(h * D, (h + 1) * D)
            gcol = gidx[h:h + 1, :].T
            G = (iota_tl == gcol).astype(jnp.float32)
            kc_parts.append(_dot_nn(G, kcol[:, sl], precision=_DEF))
            vc_parts.append(_dot_nt(G, Vn[sl, :]))
        kc = jnp.concatenate(kc_parts, axis=1)
        vc = jnp.concatenate(vc_parts, axis=1)

        qp = _bdot_nt(qc, wq_ref[...]) + bq_ref[...]
        kp = _bdot_nt(kc, wk_ref[...]) + bk_ref[...]
        vp = _bdot_nt(vc, wv_ref[...]) + bv_ref[...]

        heads = []
        for h in range(H):
            sl = slice(h * D, (h + 1) * D)
            qh = qp[:, sl]
            kh = kp[:, sl]
            vh = vp[:, sl]
            bkh = bias_k_ref[:, sl]
            bvh = bias_v_ref[:, sl]
            logits = _bdot_nt(qh, kh) * scale
            lbias = _dot_nt(qh, bkh) * scale
            m = jnp.maximum(jnp.max(logits, axis=1, keepdims=True), lbias)
            e = jnp.exp(logits - m)
            eb = jnp.exp(lbias - m)
            ssum = jnp.sum(e, axis=1, keepdims=True) + eb
            oh = (_bdot_nn(e, vh) + eb * bvh) / ssum
            heads.append(oh)
        o = jnp.concatenate(heads, axis=1)

        o = _bdot_nt(o, out_w_ref[...]) + out_b_ref[...]

        mu = jnp.mean(o, axis=1, keepdims=True)
        var = jnp.mean((o - mu) * (o - mu), axis=1, keepdims=True)
        cln = (o - mu) * jax.lax.rsqrt(var + 1e-5) * an_g_ref[...] \
            + an_b_ref[...]
        a = (cln * gamma_ref[0, 0]).T + xn

        h1 = _gn_scalar(_bdot_nn(m1_w_ref[...], a) + m1_b_ref[...],
                        m1_ng_ref[...], m1_nb_ref[...])
        h1 = jnp.maximum(h1, 0.0)
        y = _gn_scalar(_bdot_nn(m2_w_ref[...], h1) + m2_b_ref[...],
                       m2_ng_ref[...], m2_nb_ref[...])

        out_ref[bi] = a + y


def _full(shape):
    return pl.BlockSpec(shape, lambda b: (0,) * len(shape))


def _conv1x1(x, w, b):
    return jnp.einsum('bcl,oc->bol', x, w) + b[None, :, None]


def _gn3(x, g, b):
    mu = jnp.mean(x, axis=(1, 2), keepdims=True)
    var = jnp.var(x, axis=(1, 2), keepdims=True)
    return (x - mu) * jax.lax.rsqrt(var + 1e-5) * g[None, :, None] + b[None, :, None]


def _l2n(x):
    n = jnp.sqrt(jnp.sum(x * x, axis=-1, keepdims=True))
    return x / jnp.maximum(n, 1e-12)


def _ext(t):
    return t.reshape(t.shape[0], H, D, t.shape[2]).transpose(0, 1, 3, 2).reshape(t.shape[0] * H, t.shape[2], D)


@jax.jit
def kernel(x, q_w, q_b, q_ng, q_nb, kv_w, kv_b, k_ng, k_nb, v_ng, v_nb,
           in_w, in_b, bias_k, bias_v, out_w, out_b, an_g, an_b, gamma,
           m1_w, m1_b, m1_ng, m1_nb, m2_w, m2_b, m2_ng, m2_nb):
    xf = x.reshape(x.shape[0], x.shape[1], -1)
    Q = _gn3(_conv1x1(xf, q_w, q_b), q_ng, q_nb)
    KV = _conv1x1(xf, kv_w, kv_b)
    K = _gn3(KV[:, :C, :], k_ng, k_nb)
    V = _gn3(KV[:, C:, :], v_ng, v_nb)
    q = _l2n(_ext(Q))
    k = _l2n(_ext(K))
    q_probe = jnp.sum(q, axis=1)
    k_abs = jnp.abs(k) + k
    score = jnp.sum(q_probe[:, None, :] * k_abs, axis=2)

    gidx = pl.pallas_call(
        _topk_kernel,
        out_shape=jax.ShapeDtypeStruct((BH, TOPK), jnp.int32),
    )(score)
    g3 = gidx.reshape(B, H, TOPK)

    qc = q.reshape(B, H, L, D).transpose(0, 2, 1, 3).reshape(B, L, C)
    kcol = k.reshape(B, H, L, D).transpose(0, 2, 1, 3).reshape(B, L, C)

    row = lambda t: t.reshape(1, -1)
    col = lambda t: t.reshape(-1, 1)
    wq, wk, wv = in_w[:C], in_w[C:2 * C], in_w[2 * C:]
    bq, bk, bv = in_b[:C], in_b[C:2 * C], in_b[2 * C:]

    operands = [
        g3, qc, kcol, V, x,
        wq, wk, wv, row(bq), row(bk), row(bv),
        row(bias_k), row(bias_v), out_w, row(out_b),
        row(an_g), row(an_b), gamma.reshape(1, 1),
        m1_w, col(m1_b), col(m1_ng), col(m1_nb),
        m2_w, col(m2_b), col(m2_ng), col(m2_nb),
    ]
    in_specs = [
        pl.BlockSpec((BPP, H, TOPK), lambda b: (b, 0, 0)),
        pl.BlockSpec((BPP, L, C), lambda b: (b, 0, 0)),
        pl.BlockSpec((BPP, L, C), lambda b: (b, 0, 0)),
        pl.BlockSpec((BPP, C, L), lambda b: (b, 0, 0)),
        pl.BlockSpec((BPP, C, L), lambda b: (b, 0, 0)),
    ] + [_full(op.shape) for op in operands[5:]]
    out = pl.pallas_call(
        _block_kernel,
        grid=(B // BPP,),
        in_specs=in_specs,
        out_specs=pl.BlockSpec((BPP, C, L), lambda b: (b, 0, 0)),
        out_shape=jax.ShapeDtypeStruct((B, C, L), jnp.float32),
    )(*operands)
    return out


# speedup vs baseline: 1.0900x; 1.0900x over previous
import numpy as np
import jax
import jax.numpy as jnp
from jax.experimental import pallas as pl

B, C, L = 8, 384, 576
M = 1536
H = 8
TOPK = 128
D = C // H
BH = B * H

_HI = jax.lax.Precision.HIGHEST
_DEF = jax.lax.Precision.DEFAULT


def _dot_nt(a, b, precision=_DEF):
    return jax.lax.dot_general(a, b, (((1,), (1,)), ((), ())),
                               precision=precision,
                               preferred_element_type=jnp.float32)


def _dot_nn(a, b, precision=_HI):
    return jax.lax.dot_general(a, b, (((1,), (0,)), ((), ())),
                               precision=precision,
                               preferred_element_type=jnp.float32)


def _dot_nn_fast(a, b, precision=_DEF):
    return jax.lax.dot_general(a, b, (((1,), (0,)), ((), ())),
                               precision=precision,
                               preferred_element_type=jnp.float32)


def _bdot_nt(a, b):
    return jax.lax.dot_general(a.astype(jnp.bfloat16), b.astype(jnp.bfloat16),
                               (((1,), (1,)), ((), ())),
                               preferred_element_type=jnp.float32)


def _bdot_nn(a, b):
    return jax.lax.dot_general(a.astype(jnp.bfloat16), b.astype(jnp.bfloat16),
                               (((1,), (0,)), ((), ())),
                               preferred_element_type=jnp.float32)


def _gn_scalar(x, g, b):
    mu = jnp.mean(x)
    var = jnp.mean((x - mu) * (x - mu))
    return (x - mu) * jax.lax.rsqrt(var + 1e-5) * g + b


def _topk_kernel(s_ref, g_ref):
    act = s_ref[...]
    iota_l = jax.lax.broadcasted_iota(jnp.int32, (BH, L), 1)
    cols = []
    for _t in range(TOPK):
        m = jnp.max(act, axis=1, keepdims=True)
        first = jnp.min(jnp.where(act == m, iota_l, L + 1),
                        axis=1, keepdims=True)
        cols.append(first)
        act = jnp.where(iota_l == first, -jnp.inf, act)
    g_ref[...] = jnp.concatenate(cols, axis=1)


BPP = 1


def _block_kernel(g_ref, q_ref, k_ref, v_ref, xT_ref,
                  wq_ref, wk_ref, wv_ref, bq_ref, bk_ref, bv_ref,
                  bias_k_ref, bias_v_ref, out_w_ref, out_b_ref,
                  an_g_ref, an_b_ref, gamma_ref,
                  m1_w_ref, m1_b_ref, m1_ng_ref, m1_nb_ref,
                  m2_w_ref, m2_b_ref, m2_ng_ref, m2_nb_ref,
                  out_ref):
    iota_tl = jax.lax.broadcasted_iota(jnp.int32, (TOPK, L), 1)
    scale = jnp.float32(1.0 / np.sqrt(D))

    for bi in range(BPP):
        gidx = g_ref[bi]
        qc = q_ref[bi]
        kcol = k_ref[bi]
        Vn = v_ref[bi]
        xT = xT_ref[bi]

        kc_parts, vc_parts = [], []
        for h in range(H):
            sl = slice(h * D, (h + 1) * D)
            gcol = gidx[h:h + 1, :].T
            G = (iota_tl == gcol).astype(jnp.float32)
            kc_parts.append(_dot_nn(G, kcol[:, sl], precision=_DEF))
            vc_parts.append(_dot_nt(G, Vn[sl, :]))
        kc = jnp.concatenate(kc_parts, axis=1)
        vc = jnp.concatenate(vc_parts, axis=1)

        qp = _bdot_nt(qc, wq_ref[...]) + bq_ref[...]
        kp = _bdot_nt(kc, wk_ref[...]) + bk_ref[...]
        vp = _bdot_nt(vc, wv_ref[...]) + bv_ref[...]

        heads = []
        for h in range(H):
            sl = slice(h * D, (h + 1) * D)
            qh = qp[:, sl]
            kh = kp[:, sl]
            vh = vp[:, sl]
            bkh = bias_k_ref[:, sl]
            bvh = bias_v_ref[:, sl]
            logits = _bdot_nt(qh, kh) * scale
            lbias = _dot_nt(qh, bkh) * scale
            m = jnp.maximum(jnp.max(logits, axis=1, keepdims=True), lbias)
            e = jnp.exp(logits - m)
            eb = jnp.exp(lbias - m)
            ssum = jnp.sum(e, axis=1, keepdims=True) + eb
            oh = (_bdot_nn(e, vh) + eb * bvh) / ssum
            heads.append(oh)
        o = jnp.concatenate(heads, axis=1)

        o = _bdot_nt(o, out_w_ref[...]) + out_b_ref[...]

        mu = jnp.mean(o, axis=1, keepdims=True)
        var = jnp.mean((o - mu) * (o - mu), axis=1, keepdims=True)
        cln = (o - mu) * jax.lax.rsqrt(var + 1e-5) * an_g_ref[...] \
            + an_b_ref[...]
        a = cln * gamma_ref[0, 0] + xT

        h1 = _gn_scalar(_bdot_nt(a, m1_w_ref[...]) + m1_b_ref[...],
                        m1_ng_ref[...], m1_nb_ref[...])
        h1 = jnp.maximum(h1, 0.0)
        y = _gn_scalar(_bdot_nt(h1, m2_w_ref[...]) + m2_b_ref[...],
                       m2_ng_ref[...], m2_nb_ref[...])

        out_ref[bi] = a + y


def _full(shape):
    return pl.BlockSpec(shape, lambda b: (0,) * len(shape))


def _conv1x1(x, w, b):
    return jnp.einsum('bcl,oc->bol', x, w) + b[None, :, None]


def _gn3(x, g, b):
    mu = jnp.mean(x, axis=(1, 2), keepdims=True)
    var = jnp.var(x, axis=(1, 2), keepdims=True)
    return (x - mu) * jax.lax.rsqrt(var + 1e-5) * g[None, :, None] + b[None, :, None]


def _l2n(x):
    n = jnp.sqrt(jnp.sum(x * x, axis=-1, keepdims=True))
    return x / jnp.maximum(n, 1e-12)


def _ext(t):
    return t.reshape(t.shape[0], H, D, t.shape[2]).transpose(0, 1, 3, 2).reshape(t.shape[0] * H, t.shape[2], D)


@jax.jit
def kernel(x, q_w, q_b, q_ng, q_nb, kv_w, kv_b, k_ng, k_nb, v_ng, v_nb,
           in_w, in_b, bias_k, bias_v, out_w, out_b, an_g, an_b, gamma,
           m1_w, m1_b, m1_ng, m1_nb, m2_w, m2_b, m2_ng, m2_nb):
    xf = x.reshape(x.shape[0], x.shape[1], -1)
    Q = _gn3(_conv1x1(xf, q_w, q_b), q_ng, q_nb)
    KV = _conv1x1(xf, kv_w, kv_b)
    K = _gn3(KV[:, :C, :], k_ng, k_nb)
    V = _gn3(KV[:, C:, :], v_ng, v_nb)
    q = _l2n(_ext(Q))
    k = _l2n(_ext(K))
    q_probe = jnp.sum(q, axis=1)
    k_abs = jnp.abs(k) + k
    score = jnp.sum(q_probe[:, None, :] * k_abs, axis=2)

    gidx = pl.pallas_call(
        _topk_kernel,
        out_shape=jax.ShapeDtypeStruct((BH, TOPK), jnp.int32),
    )(score)
    g3 = gidx.reshape(B, H, TOPK)

    qc = q.reshape(B, H, L, D).transpose(0, 2, 1, 3).reshape(B, L, C)
    kcol = k.reshape(B, H, L, D).transpose(0, 2, 1, 3).reshape(B, L, C)
    xT = x.transpose(0, 2, 1)

    row = lambda t: t.reshape(1, -1)
    col = lambda t: t.reshape(-1, 1)
    wq, wk, wv = in_w[:C], in_w[C:2 * C], in_w[2 * C:]
    bq, bk, bv = in_b[:C], in_b[C:2 * C], in_b[2 * C:]

    operands = [
        g3, qc, kcol, V, xT,
        wq, wk, wv, row(bq), row(bk), row(bv),
        row(bias_k), row(bias_v), out_w, row(out_b),
        row(an_g), row(an_b), gamma.reshape(1, 1),
        m1_w, row(m1_b), row(m1_ng), row(m1_nb),
        m2_w, row(m2_b), row(m2_ng), row(m2_nb),
    ]
    in_specs = [
        pl.BlockSpec((BPP, H, TOPK), lambda b: (b, 0, 0)),
        pl.BlockSpec((BPP, L, C), lambda b: (b, 0, 0)),
        pl.BlockSpec((BPP, L, C), lambda b: (b, 0, 0)),
        pl.BlockSpec((BPP, C, L), lambda b: (b, 0, 0)),
        pl.BlockSpec((BPP, L, C), lambda b: (b, 0, 0)),
    ] + [_full(op.shape) for op in operands[5:]]
    out = pl.pallas_call(
        _block_kernel,
        grid=(B // BPP,),
        in_specs=in_specs,
        out_specs=pl.BlockSpec((BPP, L, C), lambda b: (b, 0, 0)),
        out_shape=jax.ShapeDtypeStruct((B, L, C), jnp.float32),
    )(*operands)
    return out.transpose(0, 2, 1)
